# baseline (speedup 1.0000x reference)
import jax
import jax.numpy as jnp
from jax.experimental import pallas as pl
from jax.experimental.pallas import tpu as pltpu

D = 1024
DFF = 4096
E = 8
R = 8
ALPHA = 8
SCALING = ALPHA / R

TM = 1024
DN = 1024

_SQRT_HALF = 0.7071067811865476


def _gelu_exact(v):
    return 0.5 * v * (1.0 + jax.lax.erf(v * _SQRT_HALF))


def _fused_body(x_ref, W1_ref, b1_ref, W2_ref, b2_ref, Wr_ref, br_ref,
                Wd_ref, Wu_ref, out_ref):
    j = pl.program_id(1)
    x = x_ref[...]
    h = _gelu_exact(
        jnp.dot(x, W1_ref[...], preferred_element_type=jnp.float32) + b1_ref[...])
    acc = jnp.dot(h, W2_ref[...], preferred_element_type=jnp.float32)

    @pl.when(j == 0)
    def _init():
        logits = jnp.dot(x, Wr_ref[...], preferred_element_type=jnp.float32) + br_ref[...]
        m = jnp.max(logits, axis=-1, keepdims=True)
        ex = jnp.exp(logits - m)
        p = ex / jnp.sum(ex, axis=-1, keepdims=True)
        eidx = jax.lax.broadcasted_iota(jnp.int32, p.shape, 1)
        p1 = jnp.max(p, axis=-1, keepdims=True)
        i1 = jnp.min(jnp.where(p == p1, eidx, E), axis=-1, keepdims=True)
        pm = jnp.where(eidx == i1, -jnp.inf, p)
        p2 = jnp.max(pm, axis=-1, keepdims=True)
        i2 = jnp.min(jnp.where(pm == p2, eidx, E), axis=-1, keepdims=True)
        denom = p1 + p2 + 1e-6
        w = (jnp.where(eidx == i1, p1 / denom, 0.0)
             + jnp.where(eidx == i2, p2 / denom, 0.0))
        rrow = jax.lax.broadcasted_iota(jnp.int32, (E, E * R), 0)
        rcol = jax.lax.broadcasted_iota(jnp.int32, (E, E * R), 1)
        rep = (rcol // R == rrow).astype(jnp.float32)
        w_rep = jnp.dot(w, rep, preferred_element_type=jnp.float32)
        a = _gelu_exact(jnp.dot(x, Wd_ref[...], preferred_element_type=jnp.float32))
        moe = jnp.dot(a * w_rep, Wu_ref[...],
                      preferred_element_type=jnp.float32) * SCALING
        out_ref[...] = moe + b2_ref[...] + acc

    @pl.when(j != 0)
    def _accum():
        out_ref[...] += acc


def kernel(x, W1, b1, W2, b2, Wr, br, w_down, w_up):
    Bq, Sq, Dq = x.shape
    T = Bq * Sq
    xf = x.reshape(T, Dq).astype(jnp.bfloat16)
    Wd = w_down.transpose(1, 0, 2).reshape(Dq, E * R).astype(jnp.bfloat16)
    Wu = w_up.reshape(E * R, Dq)
    grid = (T // TM, DFF // DN)
    out = pl.pallas_call(
        _fused_body,
        grid=grid,
        in_specs=[
            pl.BlockSpec((TM, D), lambda i, j: (i, 0)),
            pl.BlockSpec((D, DN), lambda i, j: (0, j)),
            pl.BlockSpec((1, DN), lambda i, j: (0, j)),
            pl.BlockSpec((DN, D), lambda i, j: (j, 0)),
            pl.BlockSpec((1, D), lambda i, j: (0, 0)),
            pl.BlockSpec((D, E), lambda i, j: (0, 0)),
            pl.BlockSpec((1, E), lambda i, j: (0, 0)),
            pl.BlockSpec((D, E * R), lambda i, j: (0, 0)),
            pl.BlockSpec((E * R, D), lambda i, j: (0, 0)),
        ],
        out_specs=pl.BlockSpec((TM, D), lambda i, j: (i, 0)),
        out_shape=jax.ShapeDtypeStruct((T, D), jnp.float32),
        compiler_params=pltpu.CompilerParams(
            dimension_semantics=("parallel", "arbitrary")),
    )(xf, W1.astype(jnp.bfloat16), b1.reshape(1, DFF),
      W2, b2.reshape(1, Dq), Wr.astype(jnp.bfloat16),
      br.reshape(1, E), Wd, Wu)
    return out.reshape(Bq, Sq, Dq)


# speedup vs baseline: 1.1184x; 1.1184x over previous
import jax
import jax.numpy as jnp
from jax.experimental import pallas as pl
from jax.experimental.pallas import tpu as pltpu

D = 1024
DFF = 4096
E = 8
R = 8
ALPHA = 8
SCALING = ALPHA / R

TM = 1024
DN = 1024

_SQRT_HALF = 0.7071067811865476


def _gelu_exact(v):
    return 0.5 * v * (1.0 + jax.lax.erf(v * _SQRT_HALF))


def _fused_body(x_ref, W1_ref, b1_ref, W2_ref, b2_ref, Wr_ref, br_ref,
                Wd_ref, Wu_ref, out_ref):
    j = pl.program_id(1)
    x = x_ref[...]
    h = _gelu_exact(
        jnp.dot(x, W1_ref[...], preferred_element_type=jnp.float32) + b1_ref[...])
    acc = jnp.dot(h, W2_ref[...], preferred_element_type=jnp.float32)

    @pl.when(j == 0)
    def _init():
        logits = jnp.dot(x, Wr_ref[...], preferred_element_type=jnp.float32) + br_ref[...]
        m = jnp.max(logits, axis=-1, keepdims=True)
        ex = jnp.exp(logits - m)
        p = ex / jnp.sum(ex, axis=-1, keepdims=True)
        eidx = jax.lax.broadcasted_iota(jnp.int32, p.shape, 1)
        p1 = jnp.max(p, axis=-1, keepdims=True)
        i1 = jnp.min(jnp.where(p == p1, eidx, E), axis=-1, keepdims=True)
        pm = jnp.where(eidx == i1, -jnp.inf, p)
        p2 = jnp.max(pm, axis=-1, keepdims=True)
        i2 = jnp.min(jnp.where(pm == p2, eidx, E), axis=-1, keepdims=True)
        denom = p1 + p2 + 1e-6
        w = (jnp.where(eidx == i1, p1 / denom, 0.0)
             + jnp.where(eidx == i2, p2 / denom, 0.0))
        rrow = jax.lax.broadcasted_iota(jnp.int32, (E, E * R), 0)
        rcol = jax.lax.broadcasted_iota(jnp.int32, (E, E * R), 1)
        rep = (rcol // R == rrow).astype(jnp.float32)
        w_rep = jnp.dot(w, rep, preferred_element_type=jnp.float32)
        a = _gelu_exact(jnp.dot(x, Wd_ref[...], preferred_element_type=jnp.float32))
        moe = jnp.dot(a * w_rep, Wu_ref[...],
                      preferred_element_type=jnp.float32) * SCALING
        out_ref[...] = moe + b2_ref[...] + acc

    @pl.when(j != 0)
    def _accum():
        out_ref[...] += acc


def kernel(x, W1, b1, W2, b2, Wr, br, w_down, w_up):
    Bq, Sq, Dq = x.shape
    T = Bq * Sq
    xf = x.reshape(T, Dq)
    Wd = w_down.transpose(1, 0, 2).reshape(Dq, E * R)
    Wu = w_up.reshape(E * R, Dq)
    grid = (T // TM, DFF // DN)
    out = pl.pallas_call(
        _fused_body,
        grid=grid,
        in_specs=[
            pl.BlockSpec((TM, D), lambda i, j: (i, 0)),
            pl.BlockSpec((D, DN), lambda i, j: (0, j)),
            pl.BlockSpec((1, DN), lambda i, j: (0, j)),
            pl.BlockSpec((DN, D), lambda i, j: (j, 0)),
            pl.BlockSpec((1, D), lambda i, j: (0, 0)),
            pl.BlockSpec((D, E), lambda i, j: (0, 0)),
            pl.BlockSpec((1, E), lambda i, j: (0, 0)),
            pl.BlockSpec((D, E * R), lambda i, j: (0, 0)),
            pl.BlockSpec((E * R, D), lambda i, j: (0, 0)),
        ],
        out_specs=pl.BlockSpec((TM, D), lambda i, j: (i, 0)),
        out_shape=jax.ShapeDtypeStruct((T, D), jnp.float32),
        compiler_params=pltpu.CompilerParams(
            dimension_semantics=("parallel", "arbitrary")),
    )(xf, W1, b1.reshape(1, DFF), W2, b2.reshape(1, Dq), Wr,
      br.reshape(1, E), Wd, Wu)
    return out.reshape(Bq, Sq, Dq)


# speedup vs baseline: 1.1209x; 1.0022x over previous
import jax
import jax.numpy as jnp
from jax.experimental import pallas as pl
from jax.experimental.pallas import tpu as pltpu

D = 1024
DFF = 4096
E = 8
R = 8
ALPHA = 8
SCALING = ALPHA / R

TM = 1024
DN = 1024

_SQRT_HALF = 0.7071067811865476


def _gelu_exact(v):
    return 0.5 * v * (1.0 + jax.lax.erf(v * _SQRT_HALF))


def _fused_body(x_ref, W1_ref, b1_ref, W2_ref, b2_ref, Wr_ref, br_ref,
                Wd_ref, Wu_ref, out_ref):
    j = pl.program_id(1)
    x = x_ref[...]
    h = _gelu_exact(
        jnp.dot(x, W1_ref[...], preferred_element_type=jnp.float32) + b1_ref[...])
    acc = jnp.dot(h, W2_ref[...], preferred_element_type=jnp.float32)

    @pl.when(j == 0)
    def _init():
        logits = jnp.dot(x, Wr_ref[...], preferred_element_type=jnp.float32) + br_ref[...]
        m = jnp.max(logits, axis=-1, keepdims=True)
        ex = jnp.exp(logits - m)
        p = ex / jnp.sum(ex, axis=-1, keepdims=True)
        eidx = jax.lax.broadcasted_iota(jnp.int32, p.shape, 1)
        key = jnp.bitwise_or(
            jnp.bitwise_and(jax.lax.bitcast_convert_type(p, jnp.int32), -8),
            (E - 1) - eidx)
        k1 = jnp.max(key, axis=-1, keepdims=True)
        k2 = jnp.max(jnp.where(key == k1, jnp.int32(0), key),
                     axis=-1, keepdims=True)
        sel = key >= k2
        psel = jnp.where(sel, p, 0.0)
        denom = jnp.sum(psel, axis=-1, keepdims=True) + 1e-6
        w = psel / denom
        rrow = jax.lax.broadcasted_iota(jnp.int32, (E, E * R), 0)
        rcol = jax.lax.broadcasted_iota(jnp.int32, (E, E * R), 1)
        rep = (rcol // R == rrow).astype(jnp.float32)
        w_rep = jnp.dot(w, rep, preferred_element_type=jnp.float32)
        a = _gelu_exact(jnp.dot(x, Wd_ref[...], preferred_element_type=jnp.float32))
        moe = jnp.dot(a * w_rep, Wu_ref[...],
                      preferred_element_type=jnp.float32) * SCALING
        out_ref[...] = moe + b2_ref[...] + acc

    @pl.when(j != 0)
    def _accum():
        out_ref[...] += acc


def kernel(x, W1, b1, W2, b2, Wr, br, w_down, w_up):
    Bq, Sq, Dq = x.shape
    T = Bq * Sq
    xf = x.reshape(T, Dq)
    Wd = w_down.transpose(1, 0, 2).reshape(Dq, E * R)
    Wu = w_up.reshape(E * R, Dq)
    grid = (T // TM, DFF // DN)
    out = pl.pallas_call(
        _fused_body,
        grid=grid,
        in_specs=[
            pl.BlockSpec((TM, D), lambda i, j: (i, 0)),
            pl.BlockSpec((D, DN), lambda i, j: (0, j)),
            pl.BlockSpec((1, DN), lambda i, j: (0, j)),
            pl.BlockSpec((DN, D), lambda i, j: (j, 0)),
            pl.BlockSpec((1, D), lambda i, j: (0, 0)),
            pl.BlockSpec((D, E), lambda i, j: (0, 0)),
            pl.BlockSpec((1, E), lambda i, j: (0, 0)),
            pl.BlockSpec((D, E * R), lambda i, j: (0, 0)),
            pl.BlockSpec((E * R, D), lambda i, j: (0, 0)),
        ],
        out_specs=pl.BlockSpec((TM, D), lambda i, j: (i, 0)),
        out_shape=jax.ShapeDtypeStruct((T, D), jnp.float32),
        compiler_params=pltpu.CompilerParams(
            dimension_semantics=("parallel", "arbitrary")),
    )(xf, W1, b1.reshape(1, DFF), W2, b2.reshape(1, Dq), Wr,
      br.reshape(1, E), Wd, Wu)
    return out.reshape(Bq, Sq, Dq)
